# Initial kernel scaffold; baseline (speedup 1.0000x reference)
import jax
import jax.numpy as jnp
from jax.experimental import pallas as pl


def kernel(feature, label):
    raise NotImplementedError("write your pallas kernel here")


# speedup vs baseline: 20.1313x; 20.1313x over previous
import numpy as np
import jax
import jax.numpy as jnp
from jax.experimental import pallas as pl
from jax.experimental.pallas import tpu as pltpu

_NUM_CLASSES = 19
_CH = 256
_Q = 64 * 128
_T = 16
_KS = tuple(float(k) * 0.5 for k in range(-6, 7))
_NK = len(_KS)
_TGT_NP = np.exp(-0.5 * np.asarray(_KS, np.float64) ** 2)
_TGT_NP = (_TGT_NP / _TGT_NP.sum()).astype(np.float32).reshape(1, _NK)
_CH_BLK = 64
_N_CHB = _CH // _CH_BLK
_MIN_N = 1000.0


def _hist_loss_kernel(lab_ref, feat_ref, out_ref, w_ref, acc_ref, act_ref):
    b = pl.program_id(0)
    c = pl.program_id(1)

    @pl.when(jnp.logical_and(b == 0, c == 0))
    def _prologue():
        lab = lab_ref[...]
        for cls in range(_NUM_CLASSES):
            w_ref[cls, :] = jnp.sum((lab == cls).astype(jnp.float32), axis=0)
        n_all = jnp.sum(w_ref[...], axis=1, keepdims=True)
        act_ref[0, 0] = jnp.sum((n_all > 0.0).astype(jnp.float32))
        acc_ref[0, 0] = 0.0

    cls = c + 1
    f = feat_ref[...]
    w_row = w_ref[pl.ds(cls, 1), :]
    n = jnp.sum(w_row)
    nf = jnp.maximum(n, 1.0)
    s1 = jnp.sum(f * w_row, axis=1, keepdims=True)
    s2 = jnp.sum(f * f * w_row, axis=1, keepdims=True)
    miu = s1 / nf
    var = s2 / nf - miu * miu + 1e-12
    std = jnp.sqrt(var)
    neg_half_inv_var_s = -0.5 * 25.0 / var
    cols = []
    for k in _KS:
        bin_v = miu + k * std
        t = bin_v - f
        e = jnp.exp(t * t * neg_half_inv_var_s)
        cols.append(jnp.sum(e * w_row, axis=1, keepdims=True))
    s_vals = jnp.concatenate(cols, axis=1)
    inv_rowsum = 1.0 / jnp.sum(s_vals, axis=1, keepdims=True)
    partial = 0.0
    for i in range(_NK):
        d = cols[i] * inv_rowsum - float(_TGT_NP[0, i])
        ad = jnp.abs(d)
        partial += jnp.sum(jnp.where(ad < 1.0, 0.5 * d * d, ad - 0.5))
    contrib = jnp.where(n >= _MIN_N, partial / (_CH * _NK), 0.0)
    acc_ref[0, 0] = acc_ref[0, 0] + contrib

    @pl.when(jnp.logical_and(b == _N_CHB - 1, c == _NUM_CLASSES - 2))
    def _epilogue():
        out_ref[0, 0] = acc_ref[0, 0] / act_ref[0, 0]


def kernel(feature, label):
    feat = feature.reshape(_CH, _Q)
    lab = label.reshape(256, 512).astype(jnp.int32)
    lab16 = lab.reshape(64, 4, 128, 4).transpose(1, 3, 0, 2).reshape(_T, _Q)
    out = pl.pallas_call(
        _hist_loss_kernel,
        grid=(_N_CHB, _NUM_CLASSES - 1),
        in_specs=[
            pl.BlockSpec((_T, _Q), lambda b, c: (0, 0)),
            pl.BlockSpec((_CH_BLK, _Q), lambda b, c: (b, 0)),
        ],
        out_specs=pl.BlockSpec(memory_space=pltpu.SMEM),
        out_shape=jax.ShapeDtypeStruct((1, 1), jnp.float32),
        scratch_shapes=[
            pltpu.VMEM((_NUM_CLASSES, _Q), jnp.float32),
            pltpu.SMEM((1, 1), jnp.float32),
            pltpu.SMEM((1, 1), jnp.float32),
        ],
    )(lab16, feat)
    return out.reshape(())


# speedup vs baseline: 20.2688x; 1.0068x over previous
import numpy as np
import jax
import jax.numpy as jnp
from jax.experimental import pallas as pl
from jax.experimental.pallas import tpu as pltpu

_NUM_CLASSES = 19
_NCLS_PAD = 24
_CH = 256
_Q = 64 * 128
_T = 16
_KS = tuple(float(k) * 0.5 for k in range(-6, 7))
_NK = len(_KS)
_TGT_NP = np.exp(-0.5 * np.asarray(_KS, np.float64) ** 2)
_TGT_NP = (_TGT_NP / _TGT_NP.sum()).astype(np.float32).reshape(1, _NK)
_CH_BLK = 64
_SUB = 8
_GRP = _CH_BLK // _SUB
_N_CHB = _CH // _CH_BLK
_MIN_N = 1000.0


def _prep_kernel(lab_ref, feat_ref, w8_ref, m1_ref, m2_ref, act_ref, w_ref):
    b = pl.program_id(0)

    @pl.when(b == 0)
    def _prologue():
        lab = lab_ref[...]
        w_ref[16:24, :] = jnp.zeros((8, _Q), jnp.float32)
        act = 0.0
        for cls in range(_NUM_CLASSES):
            cnt = jnp.sum((lab == cls).astype(jnp.float32), axis=0)
            w_ref[cls, :] = cnt
            act = act + (jnp.sum(cnt) > 0.0).astype(jnp.float32)
            if cls > 0:
                w8_ref[cls - 1, :, :] = jnp.broadcast_to(cnt, (_SUB, _Q))
        act_ref[0, 0] = act

    f = feat_ref[...]
    nt = (((1,), (1,)), ((), ()))
    m1_ref[...] = jax.lax.dot_general(
        f, w_ref[...], nt, preferred_element_type=jnp.float32)
    m2_ref[...] = jax.lax.dot_general(
        f * f, w_ref[...], nt, preferred_element_type=jnp.float32)


def _main_kernel(w8_ref, feat_ref, m1_ref, m2_ref, act_ref, out_ref, acc_ref):
    b = pl.program_id(0)
    c = pl.program_id(1)
    cls = c + 1

    @pl.when(jnp.logical_and(b == 0, c == 0))
    def _init():
        acc_ref[0, 0] = 0.0

    w8 = w8_ref[0]
    n = jnp.sum(w8) * (1.0 / _SUB)
    nf = jnp.maximum(n, 1.0)
    lane = jax.lax.broadcasted_iota(jnp.int32, (_CH_BLK, _NCLS_PAD), 1)
    sel = (lane == cls).astype(jnp.float32)
    m1c = jnp.sum(m1_ref[...] * sel, axis=1, keepdims=True)
    m2c = jnp.sum(m2_ref[...] * sel, axis=1, keepdims=True)
    miu = m1c / nf
    var = m2c / nf - miu * miu + 1e-12
    std = jnp.sqrt(var)
    cvar = (-0.5 * 25.0) / var
    miu3 = miu.reshape(_GRP, _SUB, 1)
    std3 = std.reshape(_GRP, _SUB, 1)
    c3 = cvar.reshape(_GRP, _SUB, 1)
    lnw = jnp.log(w8)[None]
    f3 = feat_ref[...].reshape(_GRP, _SUB, _Q)
    cols = []
    for k in _KS:
        bv = miu3 + k * std3
        t = bv - f3
        e = jnp.exp(t * t * c3 + lnw)
        cols.append(jnp.sum(e, axis=2, keepdims=True))
    s_vals = jnp.concatenate(cols, axis=2)
    inv_rs = 1.0 / jnp.sum(s_vals, axis=2, keepdims=True)
    partial = 0.0
    for i in range(_NK):
        d = cols[i] * inv_rs - float(_TGT_NP[0, i])
        ad = jnp.abs(d)
        partial += jnp.sum(jnp.where(ad < 1.0, 0.5 * d * d, ad - 0.5))
    contrib = jnp.where(n >= _MIN_N, partial / (_CH * _NK), 0.0)
    acc_ref[0, 0] = acc_ref[0, 0] + contrib

    @pl.when(jnp.logical_and(b == _N_CHB - 1, c == _NUM_CLASSES - 2))
    def _epilogue():
        out_ref[0, 0] = acc_ref[0, 0] / act_ref[0, 0]


def kernel(feature, label):
    feat = feature.reshape(_CH, _Q)
    lab = label.reshape(256, 512).astype(jnp.int32)
    lab16 = lab.reshape(64, 4, 128, 4).transpose(1, 3, 0, 2).reshape(_T, _Q)
    w8_bc, m1, m2, act = pl.pallas_call(
        _prep_kernel,
        grid=(_N_CHB,),
        in_specs=[
            pl.BlockSpec((_T, _Q), lambda b: (0, 0)),
            pl.BlockSpec((_CH_BLK, _Q), lambda b: (b, 0)),
        ],
        out_specs=[
            pl.BlockSpec((_NUM_CLASSES - 1, _SUB, _Q), lambda b: (0, 0, 0)),
            pl.BlockSpec((_CH_BLK, _NCLS_PAD), lambda b: (b, 0)),
            pl.BlockSpec((_CH_BLK, _NCLS_PAD), lambda b: (b, 0)),
            pl.BlockSpec(memory_space=pltpu.SMEM),
        ],
        out_shape=[
            jax.ShapeDtypeStruct((_NUM_CLASSES - 1, _SUB, _Q), jnp.float32),
            jax.ShapeDtypeStruct((_CH, _NCLS_PAD), jnp.float32),
            jax.ShapeDtypeStruct((_CH, _NCLS_PAD), jnp.float32),
            jax.ShapeDtypeStruct((1, 1), jnp.float32),
        ],
        scratch_shapes=[pltpu.VMEM((_NCLS_PAD, _Q), jnp.float32)],
    )(lab16, feat)
    out = pl.pallas_call(
        _main_kernel,
        grid=(_N_CHB, _NUM_CLASSES - 1),
        in_specs=[
            pl.BlockSpec((1, _SUB, _Q), lambda b, c: (c, 0, 0)),
            pl.BlockSpec((_CH_BLK, _Q), lambda b, c: (b, 0)),
            pl.BlockSpec((_CH_BLK, _NCLS_PAD), lambda b, c: (b, 0)),
            pl.BlockSpec((_CH_BLK, _NCLS_PAD), lambda b, c: (b, 0)),
            pl.BlockSpec(memory_space=pltpu.SMEM),
        ],
        out_specs=pl.BlockSpec(memory_space=pltpu.SMEM),
        out_shape=jax.ShapeDtypeStruct((1, 1), jnp.float32),
        scratch_shapes=[pltpu.SMEM((1, 1), jnp.float32)],
    )(w8_bc, feat, m1, m2, act)
    return out.reshape(())


# speedup vs baseline: 22.4666x; 1.1084x over previous
import numpy as np
import jax
import jax.numpy as jnp
from jax.experimental import pallas as pl
from jax.experimental.pallas import tpu as pltpu

_NUM_CLASSES = 19
_NCLS_PAD = 24
_CH = 256
_Q = 64 * 128
_T = 16
_KS = tuple(float(k) * 0.5 for k in range(-6, 7))
_NK = len(_KS)
_TGT_NP = np.exp(-0.5 * np.asarray(_KS, np.float64) ** 2)
_TGT_NP = (_TGT_NP / _TGT_NP.sum()).astype(np.float32).reshape(1, _NK)
_CH_BLK = 64
_SUB = 8
_GRP = _CH_BLK // _SUB
_N_CHB = _CH // _CH_BLK
_MIN_N = 1000.0


def _prep_kernel(lab_ref, feat_ref, w8_ref, m1_ref, m2_ref, act_ref, w_ref):
    b = pl.program_id(0)

    @pl.when(b == 0)
    def _prologue():
        lab = lab_ref[...]
        w_ref[16:24, :] = jnp.zeros((8, _Q), jnp.float32)
        act = 0.0
        for cls in range(_NUM_CLASSES):
            cnt = jnp.sum((lab == cls).astype(jnp.float32), axis=0)
            w_ref[cls, :] = cnt
            act = act + (jnp.sum(cnt) > 0.0).astype(jnp.float32)
            if cls > 0:
                w8_ref[cls - 1, :, :] = jnp.broadcast_to(cnt, (_SUB, _Q))
        act_ref[0, 0] = act

    f = feat_ref[...]
    nt = (((1,), (1,)), ((), ()))
    m1_ref[...] = jax.lax.dot_general(
        f, w_ref[...], nt, preferred_element_type=jnp.float32)
    m2_ref[...] = jax.lax.dot_general(
        f * f, w_ref[...], nt, preferred_element_type=jnp.float32)


def _main_kernel(w8_ref, feat_ref, m1_ref, m2_ref, act_ref, out_ref, acc_ref):
    b = pl.program_id(0)
    c = pl.program_id(1)
    cls = c + 1

    @pl.when(jnp.logical_and(b == 0, c == 0))
    def _init():
        acc_ref[0, 0] = 0.0

    w8 = w8_ref[0]
    n = jnp.sum(w8) * (1.0 / _SUB)
    nf = jnp.maximum(n, 1.0)
    lane = jax.lax.broadcasted_iota(jnp.int32, (_CH_BLK, _NCLS_PAD), 1)
    sel = (lane == cls).astype(jnp.float32)
    m1c = jnp.sum(m1_ref[...] * sel, axis=1, keepdims=True)
    m2c = jnp.sum(m2_ref[...] * sel, axis=1, keepdims=True)
    miu = m1c / nf
    var = m2c / nf - miu * miu + 1e-12
    std = jnp.sqrt(var)
    cvar = (-0.5 * 25.0) / var
    scale = jnp.sqrt(-cvar)
    miu3 = miu.reshape(_GRP, _SUB, 1)
    std3 = std.reshape(_GRP, _SUB, 1)
    s3 = scale.reshape(_GRP, _SUB, 1)
    lnw = jnp.log(w8)[None]
    f3 = feat_ref[...].reshape(_GRP, _SUB, _Q)
    fs = f3 * s3
    cols = []
    for k in _KS:
        bvs = (miu3 + k * std3) * s3
        u = bvs - fs
        e = jnp.exp(lnw - u * u)
        cols.append(jnp.sum(e, axis=2, keepdims=True))
    s_vals = jnp.concatenate(cols, axis=2)
    inv_rs = 1.0 / jnp.sum(s_vals, axis=2, keepdims=True)
    partial = 0.0
    for i in range(_NK):
        d = cols[i] * inv_rs - float(_TGT_NP[0, i])
        ad = jnp.abs(d)
        partial += jnp.sum(jnp.where(ad < 1.0, 0.5 * d * d, ad - 0.5))
    contrib = jnp.where(n >= _MIN_N, partial / (_CH * _NK), 0.0)
    acc_ref[0, 0] = acc_ref[0, 0] + contrib

    @pl.when(jnp.logical_and(b == _N_CHB - 1, c == _NUM_CLASSES - 2))
    def _epilogue():
        out_ref[0, 0] = acc_ref[0, 0] / act_ref[0, 0]


def kernel(feature, label):
    feat = feature.reshape(_CH, _Q)
    lab = label.reshape(256, 512).astype(jnp.int32)
    lab16 = lab.reshape(64, 4, 128, 4).transpose(1, 3, 0, 2).reshape(_T, _Q)
    w8_bc, m1, m2, act = pl.pallas_call(
        _prep_kernel,
        grid=(_N_CHB,),
        in_specs=[
            pl.BlockSpec((_T, _Q), lambda b: (0, 0)),
            pl.BlockSpec((_CH_BLK, _Q), lambda b: (b, 0)),
        ],
        out_specs=[
            pl.BlockSpec((_NUM_CLASSES - 1, _SUB, _Q), lambda b: (0, 0, 0)),
            pl.BlockSpec((_CH_BLK, _NCLS_PAD), lambda b: (b, 0)),
            pl.BlockSpec((_CH_BLK, _NCLS_PAD), lambda b: (b, 0)),
            pl.BlockSpec(memory_space=pltpu.SMEM),
        ],
        out_shape=[
            jax.ShapeDtypeStruct((_NUM_CLASSES - 1, _SUB, _Q), jnp.float32),
            jax.ShapeDtypeStruct((_CH, _NCLS_PAD), jnp.float32),
            jax.ShapeDtypeStruct((_CH, _NCLS_PAD), jnp.float32),
            jax.ShapeDtypeStruct((1, 1), jnp.float32),
        ],
        scratch_shapes=[pltpu.VMEM((_NCLS_PAD, _Q), jnp.float32)],
    )(lab16, feat)
    out = pl.pallas_call(
        _main_kernel,
        grid=(_N_CHB, _NUM_CLASSES - 1),
        in_specs=[
            pl.BlockSpec((1, _SUB, _Q), lambda b, c: (c, 0, 0)),
            pl.BlockSpec((_CH_BLK, _Q), lambda b, c: (b, 0)),
            pl.BlockSpec((_CH_BLK, _NCLS_PAD), lambda b, c: (b, 0)),
            pl.BlockSpec((_CH_BLK, _NCLS_PAD), lambda b, c: (b, 0)),
            pl.BlockSpec(memory_space=pltpu.SMEM),
        ],
        out_specs=pl.BlockSpec(memory_space=pltpu.SMEM),
        out_shape=jax.ShapeDtypeStruct((1, 1), jnp.float32),
        scratch_shapes=[pltpu.SMEM((1, 1), jnp.float32)],
    )(w8_bc, feat, m1, m2, act)
    return out.reshape(())


# speedup vs baseline: 26.9321x; 1.1988x over previous
import numpy as np
import jax
import jax.numpy as jnp
from jax.experimental import pallas as pl
from jax.experimental.pallas import tpu as pltpu

_NUM_CLASSES = 19
_NCLS_PAD = 24
_CH = 256
_Q = 64 * 128
_T = 16
_KS = tuple(float(k) * 0.5 for k in range(-6, 7))
_NK = len(_KS)
_TGT_NP = np.exp(-0.5 * np.asarray(_KS, np.float64) ** 2)
_TGT_NP = (_TGT_NP / _TGT_NP.sum()).astype(np.float32).reshape(1, _NK)
_CH_BLK = 128
_SUB = 8
_GRP = _CH_BLK // _SUB
_N_CHB = _CH // _CH_BLK
_MIN_N = 1000.0
_LOG2E = float(np.log2(np.e))


def _prep_kernel(lab_ref, feat_ref, w8_ref, m1_ref, m2_ref, act_ref, w_ref):
    b = pl.program_id(0)

    @pl.when(b == 0)
    def _prologue():
        lab = lab_ref[...]
        w_ref[16:24, :] = jnp.zeros((8, _Q), jnp.float32)
        act = 0.0
        for cls in range(_NUM_CLASSES):
            cnt = jnp.sum((lab == cls).astype(jnp.float32), axis=0)
            w_ref[cls, :] = cnt
            act = act + (jnp.sum(cnt) > 0.0).astype(jnp.float32)
            if cls > 0:
                w8_ref[cls - 1, :, :] = jnp.broadcast_to(cnt, (_SUB, _Q))
        act_ref[0, 0] = act

    f = feat_ref[...]
    nt = (((1,), (1,)), ((), ()))
    m1_ref[...] = jax.lax.dot_general(
        f, w_ref[...], nt, preferred_element_type=jnp.float32)
    m2_ref[...] = jax.lax.dot_general(
        f * f, w_ref[...], nt, preferred_element_type=jnp.float32)


def _main_kernel(w8_ref, feat_ref, m1_ref, m2_ref, act_ref, out_ref, acc_ref):
    b = pl.program_id(0)
    c = pl.program_id(1)
    cls = c + 1

    @pl.when(jnp.logical_and(b == 0, c == 0))
    def _init():
        acc_ref[0, 0] = 0.0

    w8 = w8_ref[0]
    n = jnp.sum(w8) * (1.0 / _SUB)
    nf = jnp.maximum(n, 1.0)
    lane = jax.lax.broadcasted_iota(jnp.int32, (_CH_BLK, _NCLS_PAD), 1)
    sel = (lane == cls).astype(jnp.float32)
    m1c = jnp.sum(m1_ref[...] * sel, axis=1, keepdims=True)
    m2c = jnp.sum(m2_ref[...] * sel, axis=1, keepdims=True)
    miu = m1c / nf
    var = m2c / nf - miu * miu + 1e-12
    std = jnp.sqrt(var)
    cvar = (-0.5 * 25.0) / var
    scale = jnp.sqrt(-cvar * _LOG2E)
    miu3 = miu.reshape(_GRP, _SUB, 1)
    std3 = std.reshape(_GRP, _SUB, 1)
    s3 = scale.reshape(_GRP, _SUB, 1)
    log2w = (jnp.log(w8) * _LOG2E)[None]
    f3 = feat_ref[...].reshape(_GRP, _SUB, _Q)
    fs = f3 * s3
    cols = []
    for k in _KS:
        bvs = (miu3 + k * std3) * s3
        u = bvs - fs
        e = jnp.exp2(log2w - u * u)
        cols.append(jnp.sum(e, axis=2, keepdims=True))
    s_vals = jnp.concatenate(cols, axis=2)
    inv_rs = 1.0 / jnp.sum(s_vals, axis=2, keepdims=True)
    partial = 0.0
    for i in range(_NK):
        d = cols[i] * inv_rs - float(_TGT_NP[0, i])
        ad = jnp.abs(d)
        partial += jnp.sum(jnp.where(ad < 1.0, 0.5 * d * d, ad - 0.5))
    contrib = jnp.where(n >= _MIN_N, partial / (_CH * _NK), 0.0)
    acc_ref[0, 0] = acc_ref[0, 0] + contrib

    @pl.when(jnp.logical_and(b == _N_CHB - 1, c == _NUM_CLASSES - 2))
    def _epilogue():
        out_ref[0, 0] = acc_ref[0, 0] / act_ref[0, 0]


def kernel(feature, label):
    feat = feature.reshape(_CH, _Q)
    lab = label.reshape(256, 512).astype(jnp.int32)
    lab16 = lab.reshape(64, 4, 128, 4).transpose(1, 3, 0, 2).reshape(_T, _Q)
    w8_bc, m1, m2, act = pl.pallas_call(
        _prep_kernel,
        grid=(_N_CHB,),
        in_specs=[
            pl.BlockSpec((_T, _Q), lambda b: (0, 0)),
            pl.BlockSpec((_CH_BLK, _Q), lambda b: (b, 0)),
        ],
        out_specs=[
            pl.BlockSpec((_NUM_CLASSES - 1, _SUB, _Q), lambda b: (0, 0, 0)),
            pl.BlockSpec((_CH_BLK, _NCLS_PAD), lambda b: (b, 0)),
            pl.BlockSpec((_CH_BLK, _NCLS_PAD), lambda b: (b, 0)),
            pl.BlockSpec(memory_space=pltpu.SMEM),
        ],
        out_shape=[
            jax.ShapeDtypeStruct((_NUM_CLASSES - 1, _SUB, _Q), jnp.float32),
            jax.ShapeDtypeStruct((_CH, _NCLS_PAD), jnp.float32),
            jax.ShapeDtypeStruct((_CH, _NCLS_PAD), jnp.float32),
            jax.ShapeDtypeStruct((1, 1), jnp.float32),
        ],
        scratch_shapes=[pltpu.VMEM((_NCLS_PAD, _Q), jnp.float32)],
    )(lab16, feat)
    out = pl.pallas_call(
        _main_kernel,
        grid=(_N_CHB, _NUM_CLASSES - 1),
        in_specs=[
            pl.BlockSpec((1, _SUB, _Q), lambda b, c: (c, 0, 0)),
            pl.BlockSpec((_CH_BLK, _Q), lambda b, c: (b, 0)),
            pl.BlockSpec((_CH_BLK, _NCLS_PAD), lambda b, c: (b, 0)),
            pl.BlockSpec((_CH_BLK, _NCLS_PAD), lambda b, c: (b, 0)),
            pl.BlockSpec(memory_space=pltpu.SMEM),
        ],
        out_specs=pl.BlockSpec(memory_space=pltpu.SMEM),
        out_shape=jax.ShapeDtypeStruct((1, 1), jnp.float32),
        scratch_shapes=[pltpu.SMEM((1, 1), jnp.float32)],
    )(w8_bc, feat, m1, m2, act)
    return out.reshape(())
